# Initial kernel scaffold; baseline (speedup 1.0000x reference)
import jax
import jax.numpy as jnp
from jax.experimental import pallas as pl


def kernel(nodes, edge_index, edge_attr, W_message, W_node, W1, b1, g1, be1, W2, b2, g2, be2, g_out, b_out):
    raise NotImplementedError("write your pallas kernel here")


# speedup vs baseline: 2.4194x; 2.4194x over previous
import functools

import jax
import jax.numpy as jnp
from jax import lax
from jax.experimental import pallas as pl
from jax.experimental.pallas import tpu as pltpu
from jax.experimental.pallas import tpu_sc as plsc

N_NODES = 10000
NODE_DIM = 128
EDGE_DIM = 16
MSG_DIM = 128
N_EDGES = 320000

NC = 2
NS = 16
NW = NC * NS
E_PER_W = N_EDGES // NW
CHUNK = 80
NCHUNK = E_PER_W // CHUNK
CP = CHUNK
NBLK = N_NODES // CP
BLK_PER_TILE = -(-NBLK // NS)


def _make_sc_agg(indirect_rows):
    mesh = plsc.VectorSubcoreMesh(core_axis_name="c", subcore_axis_name="s")

    scratch = [
        pltpu.VMEM_SHARED((N_NODES, MSG_DIM), jnp.float32),
        pltpu.VMEM((CHUNK,), jnp.int32),
        pltpu.VMEM((CHUNK,), jnp.int32),
        pltpu.VMEM((CHUNK,), jnp.int32),
        pltpu.VMEM((CHUNK,), jnp.int32),
        pltpu.VMEM((CHUNK, MSG_DIM), jnp.float32),
        pltpu.VMEM((CHUNK, MSG_DIM), jnp.float32),
        pltpu.SemaphoreType.DMA,
    ]

    @functools.partial(
        pl.kernel,
        out_type=jax.ShapeDtypeStruct((NC * N_NODES, MSG_DIM), jnp.float32),
        mesh=mesh,
        scratch_types=scratch,
    )
    def agg_kernel(table_hbm, s_hbm, r_hbm, outP,
                   acc, sidx0, sidx1, ridx0, ridx1, rows0, rows1, dsem):
        zbuf = rows0
        cid = lax.axis_index("c")
        sid = lax.axis_index("s")
        wid = cid * NS + sid

        def zero_body(i, carry):
            for k in range(MSG_DIM // 16):
                zbuf[i, pl.ds(k * 16, 16)] = jnp.zeros((16,), jnp.float32)
            return carry
        lax.fori_loop(0, CP, zero_body, 0)

        for p in range(BLK_PER_TILE):
            blk = sid + p * NS

            @pl.when(blk < NBLK)
            def _():
                for k in range(CP // 16):
                    sidx0[pl.ds(k * 16, 16)] = (
                        blk * CP + k * 16 + lax.iota(jnp.int32, 16))
                pltpu.sync_copy(zbuf, acc.at[sidx0])

        plsc.subcore_barrier()

        def do_chunk(j, sidx, ridx, rows):
            base = wid * E_PER_W + j * CHUNK
            if indirect_rows:
                pltpu.sync_copy(s_hbm.at[pl.ds(base, CHUNK)], sidx)
                pltpu.sync_copy(r_hbm.at[pl.ds(base, CHUNK)], ridx)
                pltpu.async_copy(table_hbm.at[sidx], rows, dsem).wait()
            else:
                pltpu.sync_copy(r_hbm.at[pl.ds(base, CHUNK)], ridx)
                pltpu.sync_copy(table_hbm.at[pl.ds(base, CHUNK)], rows)
            pltpu.sync_copy(rows, acc.at[ridx], add=True)

        @pl.loop(0, NCHUNK // 2)
        def edge_body(jj):
            do_chunk(jj * 2, sidx0, ridx0, rows0)
            do_chunk(jj * 2 + 1, sidx1, ridx1, rows1)

        if NCHUNK % 2:
            do_chunk(NCHUNK - 1, sidx1, ridx1, rows1)

        plsc.subcore_barrier()

        for p in range(BLK_PER_TILE):
            blk = sid + p * NS

            @pl.when(blk < NBLK)
            def _():
                for k in range(CP // 16):
                    sidx0[pl.ds(k * 16, 16)] = (
                        blk * CP + k * 16 + lax.iota(jnp.int32, 16))
                pltpu.sync_copy(acc.at[sidx0], zbuf)
                pltpu.sync_copy(
                    zbuf, outP.at[pl.ds(cid * N_NODES + blk * CP, CP)])

    return agg_kernel


def _proj_body(x_ref, w_ref, o_ref):
    o_ref[...] = jnp.dot(x_ref[...], w_ref[...],
                         preferred_element_type=jnp.float32)


def _mlp_body(x_ref, aggP_ref, aggQ_ref, Wn_ref, W1a_ref, W1b_ref,
              b1_ref, g1_ref, be1_ref, W2_ref, b2_ref, g2_ref, be2_ref,
              go_ref, bo_ref, o_ref):
    def ln(v, g, b):
        mu = jnp.mean(v, axis=-1, keepdims=True)
        var = jnp.mean((v - mu) ** 2, axis=-1, keepdims=True)
        return (v - mu) * lax.rsqrt(var + 1e-5) * g + b

    x = x_ref[...]
    agg = aggP_ref[0] + aggP_ref[1] + aggQ_ref[0] + aggQ_ref[1]
    pre1 = (jnp.dot(x, W1a_ref[...], preferred_element_type=jnp.float32)
            + jnp.dot(agg, W1b_ref[...], preferred_element_type=jnp.float32)
            + b1_ref[...])
    h = ln(jnp.maximum(pre1, 0.0), g1_ref[...], be1_ref[...])
    pre2 = jnp.dot(h, W2_ref[...], preferred_element_type=jnp.float32) + b2_ref[...]
    node_out = ln(jnp.maximum(pre2, 0.0), g2_ref[...], be2_ref[...])
    out = ln(jnp.dot(x, Wn_ref[...], preferred_element_type=jnp.float32)
             + node_out, go_ref[...], bo_ref[...])
    o_ref[...] = out


def kernel(nodes, edge_index, edge_attr, W_message, W_node,
           W1, b1, g1, be1, W2, b2, g2, be2, g_out, b_out):
    senders = edge_index[0].astype(jnp.int32)
    receivers = edge_index[1].astype(jnp.int32)
    WmX = W_message[:NODE_DIM]
    WmE = W_message[NODE_DIM:]
    W1a = W1[:NODE_DIM]
    W1b = W1[NODE_DIM:]

    RB = 1000
    grid = N_NODES // RB

    P = pl.pallas_call(
        _proj_body,
        grid=(grid,),
        in_specs=[
            pl.BlockSpec((RB, NODE_DIM), lambda i: (i, 0)),
            pl.BlockSpec((NODE_DIM, MSG_DIM), lambda i: (0, 0)),
        ],
        out_specs=pl.BlockSpec((RB, MSG_DIM), lambda i: (i, 0)),
        out_shape=jax.ShapeDtypeStruct((N_NODES, MSG_DIM), jnp.float32),
    )(nodes, WmX)

    EB = 4000
    Q = pl.pallas_call(
        _proj_body,
        grid=(N_EDGES // EB,),
        in_specs=[
            pl.BlockSpec((EB, EDGE_DIM), lambda i: (i, 0)),
            pl.BlockSpec((EDGE_DIM, MSG_DIM), lambda i: (0, 0)),
        ],
        out_specs=pl.BlockSpec((EB, MSG_DIM), lambda i: (i, 0)),
        out_shape=jax.ShapeDtypeStruct((N_EDGES, MSG_DIM), jnp.float32),
    )(edge_attr, WmE)

    aggP = _make_sc_agg(True)(P, senders, receivers)
    aggQ = _make_sc_agg(False)(Q, senders, receivers)
    aggP = aggP.reshape(NC, N_NODES, MSG_DIM)
    aggQ = aggQ.reshape(NC, N_NODES, MSG_DIM)

    vec = lambda v: v.reshape(1, NODE_DIM)
    full = lambda r, c: pl.BlockSpec((r, c), lambda i: (0, 0))
    out = pl.pallas_call(
        _mlp_body,
        grid=(grid,),
        in_specs=[
            pl.BlockSpec((RB, NODE_DIM), lambda i: (i, 0)),
            pl.BlockSpec((NC, RB, MSG_DIM), lambda i: (0, i, 0)),
            pl.BlockSpec((NC, RB, MSG_DIM), lambda i: (0, i, 0)),
            full(NODE_DIM, NODE_DIM),
            full(NODE_DIM, NODE_DIM),
            full(MSG_DIM, NODE_DIM),
            full(1, NODE_DIM), full(1, NODE_DIM), full(1, NODE_DIM),
            full(NODE_DIM, NODE_DIM),
            full(1, NODE_DIM), full(1, NODE_DIM), full(1, NODE_DIM),
            full(1, NODE_DIM), full(1, NODE_DIM),
        ],
        out_specs=pl.BlockSpec((RB, NODE_DIM), lambda i: (i, 0)),
        out_shape=jax.ShapeDtypeStruct((N_NODES, NODE_DIM), jnp.float32),
    )(nodes, aggP, aggQ, W_node, W1a, W1b,
      vec(b1), vec(g1), vec(be1), W2, vec(b2), vec(g2), vec(be2),
      vec(g_out), vec(b_out))
    return out
